# Initial kernel scaffold; baseline (speedup 1.0000x reference)
import functools
import math

import jax
import jax.numpy as jnp
import numpy as np
from jax.experimental import pallas as pl
from jax.experimental.pallas import tpu as pltpu

N_NODES = 4096
N_EDGES = 131072
HIDDEN = 128
NUM_CLASSES = 8


def _build_constants():
    N = N_NODES
    n = np.arange(N)
    C = -2.0 * np.eye(N) + np.eye(N, k=1) + np.eye(N, k=-1)
    C[0, N - 1] += 1.0
    C[N - 1, 0] += 1.0
    S = C + np.diag(2.0 * np.cos(2.0 * np.pi * n / N) - 2.0)
    _, evecs = np.linalg.eigh(S)
    E = np.ascontiguousarray(evecs[:, ::-1]).astype(np.float32)
    idxv = np.concatenate([np.arange(N - 1), [N if N % 2 == 0 else N - 1]]).astype(
        np.float32
    )
    cut = max(1, int(round(0.15 * N)))
    sel = np.concatenate([np.arange(cut), np.arange(N - cut, N)])
    R = E[sel, :]
    RPAD = 1280
    Rp = np.zeros((RPAD, N), np.float32)
    Rp[: R.shape[0]] = R
    ET = np.ascontiguousarray(E.T)
    RT = np.ascontiguousarray(Rp.T)
    return E, ET, Rp, RT, idxv, RPAD


_E, _ET, _R, _RT, _IDXV, _RPAD = _build_constants()


_BLK = 512


def _mv_kern(a_ref, b_ref, o_ref):
    o_ref[...] = jax.lax.dot_general(
        a_ref[...], b_ref[...], (((1,), (0,)), ((), ())),
        preferred_element_type=jnp.float32)


def _matmul(a, b, blk):
    m, k = a.shape
    _, nn = b.shape
    return pl.pallas_call(
        _mv_kern,
        grid=(m // blk,),
        in_specs=[
            pl.BlockSpec((blk, k), lambda i: (i, 0)),
            pl.BlockSpec((k, nn), lambda i: (0, 0)),
        ],
        out_specs=pl.BlockSpec((blk, nn), lambda i: (i, 0)),
        out_shape=jax.ShapeDtypeStruct((m, nn), jnp.float32),
    )(a, b)


def _s2_kern(r_ref, t_ref, c_ref, s_ref, o_ref):
    t = t_ref[...]
    s = jnp.concatenate([t * c_ref[...], -t * s_ref[...]], axis=1)
    o_ref[...] = jax.lax.dot_general(
        r_ref[...], s, (((1,), (0,)), ((), ())),
        preferred_element_type=jnp.float32)


def _s4_kern(e_ref, u_ref, c_ref, s_ref, o_ref):
    u = u_ref[...]
    c = c_ref[...]
    s = s_ref[...]
    u0 = u[:, 0:1]
    u1 = u[:, 1:2]
    q = jnp.concatenate([u0 * c - u1 * s, u0 * s + u1 * c], axis=1)
    ya = jax.lax.dot_general(
        e_ref[...], q, (((1,), (0,)), ((), ())),
        preferred_element_type=jnp.float32)
    o_ref[...] = jnp.sqrt(ya[:, 0:1] * ya[:, 0:1] + ya[:, 1:2] * ya[:, 1:2])


def _frft_y(x, order):
    N = N_NODES
    E = jnp.asarray(_E)
    ET = jnp.asarray(_ET)
    R = jnp.asarray(_R)
    RT = jnp.asarray(_RT)
    phi = ((math.pi / 2.0) * order) * jnp.asarray(_IDXV)
    c = jnp.cos(phi).reshape(N, 1)
    s = jnp.sin(phi).reshape(N, 1)

    t = _matmul(ET, x, _BLK)
    w = pl.pallas_call(
        _s2_kern,
        grid=(_RPAD // 320,),
        in_specs=[
            pl.BlockSpec((320, N), lambda i: (i, 0)),
            pl.BlockSpec((N, 1), lambda i: (0, 0)),
            pl.BlockSpec((N, 1), lambda i: (0, 0)),
            pl.BlockSpec((N, 1), lambda i: (0, 0)),
        ],
        out_specs=pl.BlockSpec((320, 2), lambda i: (i, 0)),
        out_shape=jax.ShapeDtypeStruct((_RPAD, 2), jnp.float32),
    )(R, t, c, s)
    u = _matmul(RT, w, _BLK)
    y = pl.pallas_call(
        _s4_kern,
        grid=(N // _BLK,),
        in_specs=[
            pl.BlockSpec((_BLK, N), lambda i: (i, 0)),
            pl.BlockSpec((N, 2), lambda i: (0, 0)),
            pl.BlockSpec((_BLK, 1), lambda i: (i, 0)),
            pl.BlockSpec((_BLK, 1), lambda i: (i, 0)),
        ],
        out_specs=pl.BlockSpec((_BLK, 1), lambda i: (i, 0)),
        out_shape=jax.ShapeDtypeStruct((N, 1), jnp.float32),
    )(E, u, c, s)
    return y.reshape(N)


def kernel(x, edge_index, order, conv1_w, conv1_b, conv2_w, conv2_b, conv3_w,
           conv3_b, gn1_w, gn1_b, gn1_ms, gn2_w, gn2_b, gn2_ms, gn3_w, gn3_b,
           gn3_ms, lin_w, lin_b):
    N = N_NODES
    Y = _frft_y(x, order)

    row, col = edge_index[0], edge_index[1]
    d = jnp.abs(Y[row] - Y[col]) / (jnp.abs(Y[row]) + jnp.abs(Y[col]) + 1e-8)
    ew = jnp.clip(1.0 - d, 1e-6, None)
    deg = jnp.zeros((N,), jnp.float32).at[row].add(ew)
    dis = jnp.where(deg > 0, 1.0 / jnp.sqrt(jnp.maximum(deg, 1e-12)), 0.0)
    lapw = -dis[row] * ew * dis[col]

    def lmv(v):
        return jnp.zeros((N, v.shape[1]), v.dtype).at[col].add(lapw[:, None] * v[row])

    def cheb(h, W, b):
        Tx0 = h
        Tx1 = lmv(Tx0)
        Tx2 = 2.0 * lmv(Tx1) - Tx0
        return Tx0 @ W[0] + Tx1 @ W[1] + Tx2 @ W[2] + b

    def gnorm(h, w, b, ms):
        mean = jnp.mean(h, axis=0, keepdims=True)
        out = h - ms * mean
        var = jnp.mean(out * out, axis=0, keepdims=True)
        return w * out / jnp.sqrt(var + 1e-5) + b

    h = jax.nn.relu(gnorm(cheb(x, conv1_w, conv1_b), gn1_w, gn1_b, gn1_ms))
    h = jax.nn.relu(gnorm(cheb(h, conv2_w, conv2_b), gn2_w, gn2_b, gn2_ms))
    h = jax.nn.relu(gnorm(cheb(h, conv3_w, conv3_b), gn3_w, gn3_b, gn3_ms))
    gpool = jnp.max(h, axis=0, keepdims=True)
    return gpool @ lin_w + lin_b


# speedup vs baseline: 1.0026x; 1.0026x over previous
import functools
import math

import jax
import jax.numpy as jnp
import numpy as np
from jax.experimental import pallas as pl
from jax.experimental.pallas import tpu as pltpu

N_NODES = 4096
N_EDGES = 131072
HIDDEN = 128
NUM_CLASSES = 8


def _build_constants():
    N = N_NODES
    n = np.arange(N)
    C = -2.0 * np.eye(N) + np.eye(N, k=1) + np.eye(N, k=-1)
    C[0, N - 1] += 1.0
    C[N - 1, 0] += 1.0
    S = C + np.diag(2.0 * np.cos(2.0 * np.pi * n / N) - 2.0)
    _, evecs = np.linalg.eigh(S)
    E = np.ascontiguousarray(evecs[:, ::-1]).astype(np.float32)
    idxv = np.concatenate([np.arange(N - 1), [N if N % 2 == 0 else N - 1]]).astype(
        np.float32
    )
    cut = max(1, int(round(0.15 * N)))
    sel = np.concatenate([np.arange(cut), np.arange(N - cut, N)])
    R = E[sel, :]
    RPAD = 1280
    Rp = np.zeros((RPAD, N), np.float32)
    Rp[: R.shape[0]] = R
    ET = np.ascontiguousarray(E.T)
    RT = np.ascontiguousarray(Rp.T)
    return E, ET, Rp, RT, idxv, RPAD


_E, _ET, _R, _RT, _IDXV, _RPAD = _build_constants()


_BLK = 512


def _mv_kern(a_ref, b_ref, o_ref):
    o_ref[...] = jax.lax.dot_general(
        a_ref[...], b_ref[...], (((1,), (0,)), ((), ())),
        preferred_element_type=jnp.float32)


def _matmul(a, b, blk):
    m, k = a.shape
    _, nn = b.shape
    return pl.pallas_call(
        _mv_kern,
        grid=(m // blk,),
        in_specs=[
            pl.BlockSpec((blk, k), lambda i: (i, 0)),
            pl.BlockSpec((k, nn), lambda i: (0, 0)),
        ],
        out_specs=pl.BlockSpec((blk, nn), lambda i: (i, 0)),
        out_shape=jax.ShapeDtypeStruct((m, nn), jnp.float32),
    )(a, b)


def _s2_kern(r_ref, t_ref, c_ref, s_ref, o_ref):
    t = t_ref[...]
    s = jnp.concatenate([t * c_ref[...], -t * s_ref[...]], axis=1)
    o_ref[...] = jax.lax.dot_general(
        r_ref[...], s, (((1,), (0,)), ((), ())),
        preferred_element_type=jnp.float32)


def _s4_kern(e_ref, u_ref, c_ref, s_ref, o_ref):
    u = u_ref[...]
    c = c_ref[...]
    s = s_ref[...]
    u0 = u[:, 0:1]
    u1 = u[:, 1:2]
    q = jnp.concatenate([u0 * c - u1 * s, u0 * s + u1 * c], axis=1)
    ya = jax.lax.dot_general(
        e_ref[...], q, (((1,), (0,)), ((), ())),
        preferred_element_type=jnp.float32)
    o_ref[...] = jnp.sqrt(ya[:, 0:1] * ya[:, 0:1] + ya[:, 1:2] * ya[:, 1:2])


def _frft_y(x, order):
    N = N_NODES
    E = jnp.asarray(_E)
    ET = jnp.asarray(_ET)
    R = jnp.asarray(_R)
    RT = jnp.asarray(_RT)
    phi = ((math.pi / 2.0) * order) * jnp.asarray(_IDXV)
    c = jnp.cos(phi).reshape(N, 1)
    s = jnp.sin(phi).reshape(N, 1)

    t = _matmul(ET, x, _BLK)
    w = pl.pallas_call(
        _s2_kern,
        grid=(_RPAD // 320,),
        in_specs=[
            pl.BlockSpec((320, N), lambda i: (i, 0)),
            pl.BlockSpec((N, 1), lambda i: (0, 0)),
            pl.BlockSpec((N, 1), lambda i: (0, 0)),
            pl.BlockSpec((N, 1), lambda i: (0, 0)),
        ],
        out_specs=pl.BlockSpec((320, 2), lambda i: (i, 0)),
        out_shape=jax.ShapeDtypeStruct((_RPAD, 2), jnp.float32),
    )(R, t, c, s)
    u = _matmul(RT, w, _BLK)
    y = pl.pallas_call(
        _s4_kern,
        grid=(N // _BLK,),
        in_specs=[
            pl.BlockSpec((_BLK, N), lambda i: (i, 0)),
            pl.BlockSpec((N, 2), lambda i: (0, 0)),
            pl.BlockSpec((N, 1), lambda i: (0, 0)),
            pl.BlockSpec((N, 1), lambda i: (0, 0)),
        ],
        out_specs=pl.BlockSpec((_BLK, 1), lambda i: (i, 0)),
        out_shape=jax.ShapeDtypeStruct((N, 1), jnp.float32),
    )(E, u, c, s)
    return y.reshape(N)


def kernel(x, edge_index, order, conv1_w, conv1_b, conv2_w, conv2_b, conv3_w,
           conv3_b, gn1_w, gn1_b, gn1_ms, gn2_w, gn2_b, gn2_ms, gn3_w, gn3_b,
           gn3_ms, lin_w, lin_b):
    N = N_NODES
    Y = _frft_y(x, order)

    row, col = edge_index[0], edge_index[1]
    d = jnp.abs(Y[row] - Y[col]) / (jnp.abs(Y[row]) + jnp.abs(Y[col]) + 1e-8)
    ew = jnp.clip(1.0 - d, 1e-6, None)
    deg = jnp.zeros((N,), jnp.float32).at[row].add(ew)
    dis = jnp.where(deg > 0, 1.0 / jnp.sqrt(jnp.maximum(deg, 1e-12)), 0.0)
    lapw = -dis[row] * ew * dis[col]

    def lmv(v):
        return jnp.zeros((N, v.shape[1]), v.dtype).at[col].add(lapw[:, None] * v[row])

    def cheb(h, W, b):
        Tx0 = h
        Tx1 = lmv(Tx0)
        Tx2 = 2.0 * lmv(Tx1) - Tx0
        return Tx0 @ W[0] + Tx1 @ W[1] + Tx2 @ W[2] + b

    def gnorm(h, w, b, ms):
        mean = jnp.mean(h, axis=0, keepdims=True)
        out = h - ms * mean
        var = jnp.mean(out * out, axis=0, keepdims=True)
        return w * out / jnp.sqrt(var + 1e-5) + b

    h = jax.nn.relu(gnorm(cheb(x, conv1_w, conv1_b), gn1_w, gn1_b, gn1_ms))
    h = jax.nn.relu(gnorm(cheb(h, conv2_w, conv2_b), gn2_w, gn2_b, gn2_ms))
    h = jax.nn.relu(gnorm(cheb(h, conv3_w, conv3_b), gn3_w, gn3_b, gn3_ms))
    gpool = jnp.max(h, axis=0, keepdims=True)
    return gpool @ lin_w + lin_b


# speedup vs baseline: 15.4646x; 15.4248x over previous
import functools
import math

import jax
import jax.numpy as jnp
import numpy as np
from jax import lax
from jax.experimental import pallas as pl
from jax.experimental.pallas import tpu as pltpu
from jax.experimental.pallas import tpu_sc as plsc

N_NODES = 4096
N_EDGES = 131072
HIDDEN = 128
NUM_CLASSES = 8

NW = 32
EPW = N_EDGES // NW
CHUNK = 256
NPS = N_NODES // 16


def _build_constants():
    N = N_NODES
    n = np.arange(N)
    C = -2.0 * np.eye(N) + np.eye(N, k=1) + np.eye(N, k=-1)
    C[0, N - 1] += 1.0
    C[N - 1, 0] += 1.0
    S = C + np.diag(2.0 * np.cos(2.0 * np.pi * n / N) - 2.0)
    _, evecs = np.linalg.eigh(S)
    E = np.ascontiguousarray(evecs[:, ::-1]).astype(np.float32)
    idxv = np.concatenate([np.arange(N - 1), [N if N % 2 == 0 else N - 1]]).astype(
        np.float32
    )
    cut = max(1, int(round(0.15 * N)))
    sel = np.concatenate([np.arange(cut), np.arange(N - cut, N)])
    R = E[sel, :]
    RPAD = 1280
    Rp = np.zeros((RPAD, N), np.float32)
    Rp[: R.shape[0]] = R
    ET = np.ascontiguousarray(E.T)
    RT = np.ascontiguousarray(Rp.T)
    return E, ET, Rp, RT, idxv, RPAD


_E, _ET, _R, _RT, _IDXV, _RPAD = _build_constants()


_BLK = 512


def _mv_kern(a_ref, b_ref, o_ref):
    o_ref[...] = jax.lax.dot_general(
        a_ref[...], b_ref[...], (((1,), (0,)), ((), ())),
        preferred_element_type=jnp.float32)


def _matmul(a, b, blk):
    m, k = a.shape
    _, nn = b.shape
    return pl.pallas_call(
        _mv_kern,
        grid=(m // blk,),
        in_specs=[
            pl.BlockSpec((blk, k), lambda i: (i, 0)),
            pl.BlockSpec((k, nn), lambda i: (0, 0)),
        ],
        out_specs=pl.BlockSpec((blk, nn), lambda i: (i, 0)),
        out_shape=jax.ShapeDtypeStruct((m, nn), jnp.float32),
    )(a, b)


def _s2_kern(r_ref, t_ref, c_ref, s_ref, o_ref):
    t = t_ref[...]
    s = jnp.concatenate([t * c_ref[...], -t * s_ref[...]], axis=1)
    o_ref[...] = jax.lax.dot_general(
        r_ref[...], s, (((1,), (0,)), ((), ())),
        preferred_element_type=jnp.float32)


def _s4_kern(e_ref, u_ref, c_ref, s_ref, o_ref):
    u = u_ref[...]
    c = c_ref[...]
    s = s_ref[...]
    u0 = u[:, 0:1]
    u1 = u[:, 1:2]
    q = jnp.concatenate([u0 * c - u1 * s, u0 * s + u1 * c], axis=1)
    ya = jax.lax.dot_general(
        e_ref[...], q, (((1,), (0,)), ((), ())),
        preferred_element_type=jnp.float32)
    o_ref[...] = jnp.sqrt(ya[:, 0:1] * ya[:, 0:1] + ya[:, 1:2] * ya[:, 1:2])


def _frft_y(x, order):
    N = N_NODES
    E = jnp.asarray(_E)
    ET = jnp.asarray(_ET)
    R = jnp.asarray(_R)
    RT = jnp.asarray(_RT)
    phi = ((math.pi / 2.0) * order) * jnp.asarray(_IDXV)
    c = jnp.cos(phi).reshape(N, 1)
    s = jnp.sin(phi).reshape(N, 1)

    t = _matmul(ET, x, _BLK)
    w = pl.pallas_call(
        _s2_kern,
        grid=(_RPAD // 320,),
        in_specs=[
            pl.BlockSpec((320, N), lambda i: (i, 0)),
            pl.BlockSpec((N, 1), lambda i: (0, 0)),
            pl.BlockSpec((N, 1), lambda i: (0, 0)),
            pl.BlockSpec((N, 1), lambda i: (0, 0)),
        ],
        out_specs=pl.BlockSpec((320, 2), lambda i: (i, 0)),
        out_shape=jax.ShapeDtypeStruct((_RPAD, 2), jnp.float32),
    )(R, t, c, s)
    u = _matmul(RT, w, _BLK)
    y = pl.pallas_call(
        _s4_kern,
        grid=(N // _BLK,),
        in_specs=[
            pl.BlockSpec((_BLK, N), lambda i: (i, 0)),
            pl.BlockSpec((N, 2), lambda i: (0, 0)),
            pl.BlockSpec((N, 1), lambda i: (0, 0)),
            pl.BlockSpec((N, 1), lambda i: (0, 0)),
        ],
        out_specs=pl.BlockSpec((_BLK, 1), lambda i: (i, 0)),
        out_shape=jax.ShapeDtypeStruct((N, 1), jnp.float32),
    )(E, u, c, s)
    return y.reshape(N)


_MESH = plsc.VectorSubcoreMesh(core_axis_name="c", subcore_axis_name="s")
_SC_PARAMS = pltpu.CompilerParams(needs_layout_passes=False)


def _wid():
    return lax.axis_index("c") * 16 + lax.axis_index("s")


def _lane_iota():
    return lax.iota(jnp.int32, 16)


def _zero_lanes(lanes_v):
    z = jnp.zeros((16,), jnp.float32)

    def body(i, _):
        for l in range(16):
            lanes_v[l, pl.ds(i * 16, 16)] = z
        return 0

    lax.fori_loop(0, N_NODES // 16, body, 0)


def _reduce_lanes(lanes_v, red_v):
    def body(i, _):
        acc = lanes_v[0, pl.ds(i * 16, 16)]
        for l in range(1, 16):
            acc = acc + lanes_v[l, pl.ds(i * 16, 16)]
        red_v[pl.ds(i * 16, 16)] = acc
        return 0

    lax.fori_loop(0, N_NODES // 16, body, 0)


@functools.partial(
    pl.kernel,
    out_type=[
        jax.ShapeDtypeStruct((N_EDGES,), jnp.float32),
        jax.ShapeDtypeStruct((NW, N_NODES), jnp.float32),
    ],
    mesh=_MESH,
    compiler_params=_SC_PARAMS,
    scratch_types=[
        pltpu.VMEM((N_NODES,), jnp.float32),
        pltpu.VMEM((EPW,), jnp.int32),
        pltpu.VMEM((EPW,), jnp.int32),
        pltpu.VMEM((EPW,), jnp.float32),
        pltpu.VMEM((16, N_NODES), jnp.float32),
        pltpu.VMEM((N_NODES,), jnp.float32),
    ],
)
def _k1(y_hbm, row_hbm, col_hbm, ew_hbm, degp_hbm,
        y_v, row_v, col_v, ew_v, lanes_v, red_v):
    w = _wid()
    base = w * EPW
    pltpu.sync_copy(y_hbm, y_v)
    pltpu.sync_copy(row_hbm.at[pl.ds(base, EPW)], row_v)
    pltpu.sync_copy(col_hbm.at[pl.ds(base, EPW)], col_v)
    _zero_lanes(lanes_v)
    li = _lane_iota()

    def body(g, _):
        r16 = row_v[pl.ds(g * 16, 16)]
        c16 = col_v[pl.ds(g * 16, 16)]
        yr = plsc.load_gather(y_v, [r16])
        yc = plsc.load_gather(y_v, [c16])
        d = jnp.abs(yr - yc) / (jnp.abs(yr) + jnp.abs(yc) + 1e-8)
        ew16 = jnp.maximum(1.0 - d, 1e-6)
        ew_v[pl.ds(g * 16, 16)] = ew16
        plsc.addupdate_scatter(lanes_v, [li, r16], ew16)
        return 0

    lax.fori_loop(0, EPW // 16, body, 0)
    _reduce_lanes(lanes_v, red_v)
    pltpu.sync_copy(ew_v, ew_hbm.at[pl.ds(base, EPW)])
    pltpu.sync_copy(red_v, degp_hbm.at[w])


@functools.partial(
    pl.kernel,
    out_type=[
        jax.ShapeDtypeStruct((N_EDGES,), jnp.float32),
        jax.ShapeDtypeStruct((NW, N_NODES), jnp.float32),
    ],
    mesh=_MESH,
    compiler_params=_SC_PARAMS,
    scratch_types=[
        pltpu.VMEM((N_NODES,), jnp.float32),
        pltpu.VMEM((N_NODES,), jnp.float32),
        pltpu.VMEM((EPW,), jnp.int32),
        pltpu.VMEM((EPW,), jnp.int32),
        pltpu.VMEM((EPW,), jnp.float32),
        pltpu.VMEM((EPW,), jnp.float32),
        pltpu.VMEM((16, N_NODES), jnp.float32),
        pltpu.VMEM((N_NODES,), jnp.float32),
    ],
)
def _k2(dis_hbm, x_hbm, ew_hbm, row_hbm, col_hbm, lapw_hbm, t1p_hbm,
        dis_v, x_v, row_v, col_v, ew_v, lapw_v, lanes_v, red_v):
    w = _wid()
    base = w * EPW
    pltpu.sync_copy(dis_hbm, dis_v)
    pltpu.sync_copy(x_hbm, x_v)
    pltpu.sync_copy(row_hbm.at[pl.ds(base, EPW)], row_v)
    pltpu.sync_copy(col_hbm.at[pl.ds(base, EPW)], col_v)
    pltpu.sync_copy(ew_hbm.at[pl.ds(base, EPW)], ew_v)
    _zero_lanes(lanes_v)
    li = _lane_iota()

    def body(g, _):
        r16 = row_v[pl.ds(g * 16, 16)]
        c16 = col_v[pl.ds(g * 16, 16)]
        ew16 = ew_v[pl.ds(g * 16, 16)]
        dr = plsc.load_gather(dis_v, [r16])
        dc = plsc.load_gather(dis_v, [c16])
        lw16 = -dr * ew16 * dc
        lapw_v[pl.ds(g * 16, 16)] = lw16
        xr = plsc.load_gather(x_v, [r16])
        plsc.addupdate_scatter(lanes_v, [li, c16], lw16 * xr)
        return 0

    lax.fori_loop(0, EPW // 16, body, 0)
    _reduce_lanes(lanes_v, red_v)
    pltpu.sync_copy(lapw_v, lapw_hbm.at[pl.ds(base, EPW)])
    pltpu.sync_copy(red_v, t1p_hbm.at[w])


@functools.partial(
    pl.kernel,
    out_type=jax.ShapeDtypeStruct((NW, N_NODES), jnp.float32),
    mesh=_MESH,
    compiler_params=_SC_PARAMS,
    scratch_types=[
        pltpu.VMEM((N_NODES,), jnp.float32),
        pltpu.VMEM((EPW,), jnp.int32),
        pltpu.VMEM((EPW,), jnp.int32),
        pltpu.VMEM((EPW,), jnp.float32),
        pltpu.VMEM((16, N_NODES), jnp.float32),
        pltpu.VMEM((N_NODES,), jnp.float32),
    ],
)
def _k3(t1_hbm, lapw_hbm, row_hbm, col_hbm, t2p_hbm,
        v_v, row_v, col_v, lapw_v, lanes_v, red_v):
    w = _wid()
    base = w * EPW
    pltpu.sync_copy(t1_hbm, v_v)
    pltpu.sync_copy(row_hbm.at[pl.ds(base, EPW)], row_v)
    pltpu.sync_copy(col_hbm.at[pl.ds(base, EPW)], col_v)
    pltpu.sync_copy(lapw_hbm.at[pl.ds(base, EPW)], lapw_v)
    _zero_lanes(lanes_v)
    li = _lane_iota()

    def body(g, _):
        r16 = row_v[pl.ds(g * 16, 16)]
        c16 = col_v[pl.ds(g * 16, 16)]
        lw16 = lapw_v[pl.ds(g * 16, 16)]
        vr = plsc.load_gather(v_v, [r16])
        plsc.addupdate_scatter(lanes_v, [li, c16], lw16 * vr)
        return 0

    lax.fori_loop(0, EPW // 16, body, 0)
    _reduce_lanes(lanes_v, red_v)
    pltpu.sync_copy(red_v, t2p_hbm.at[w])


@functools.partial(
    pl.kernel,
    out_type=jax.ShapeDtypeStruct((2, N_NODES, HIDDEN), jnp.float32),
    mesh=_MESH,
    compiler_params=_SC_PARAMS,
    scratch_types=[
        pltpu.VMEM((CHUNK,), jnp.int32),
        pltpu.VMEM((CHUNK,), jnp.int32),
        pltpu.VMEM((CHUNK,), jnp.float32),
        pltpu.VMEM((CHUNK, HIDDEN), jnp.float32),
        pltpu.VMEM_SHARED((N_NODES, HIDDEN), jnp.float32),
        pltpu.SemaphoreType.DMA,
    ],
)
def _k4(h_hbm, lapw_hbm, row_hbm, col_hbm, out_hbm,
        row_v, col_v, lw_v, rows_v, acc_sh, sem):
    cid = lax.axis_index("c")
    sid = lax.axis_index("s")
    w = cid * 16 + sid
    zrow = jnp.zeros((16,), jnp.float32)

    def zbody(i, _):
        for j in range(8):
            rows_v[i, pl.ds(j * 16, 16)] = zrow
        return 0

    lax.fori_loop(0, CHUNK, zbody, 0)
    pltpu.sync_copy(rows_v, acc_sh.at[pl.ds(sid * NPS, NPS)])
    plsc.subcore_barrier()

    def chunk_body(g, _):
        base = w * EPW + g * CHUNK
        pltpu.sync_copy(row_hbm.at[pl.ds(base, CHUNK)], row_v)
        pltpu.sync_copy(col_hbm.at[pl.ds(base, CHUNK)], col_v)
        pltpu.sync_copy(lapw_hbm.at[pl.ds(base, CHUNK)], lw_v)
        pltpu.async_copy(h_hbm.at[row_v], rows_v, sem).wait()

        def scale_body(e, _):
            lw = plsc.load_gather(lw_v, [jnp.zeros((16,), jnp.int32) + e])
            for j in range(8):
                rows_v[e, pl.ds(j * 16, 16)] = rows_v[e, pl.ds(j * 16, 16)] * lw
            return 0

        lax.fori_loop(0, CHUNK, scale_body, 0)
        pltpu.sync_copy(rows_v, acc_sh.at[col_v], add=True)
        return 0

    lax.fori_loop(0, EPW // CHUNK, chunk_body, 0)
    plsc.subcore_barrier()
    pltpu.sync_copy(acc_sh.at[pl.ds(sid * NPS, NPS)],
                    out_hbm.at[cid, pl.ds(sid * NPS, NPS)])


def _dis_kern(degp_ref, o_ref):
    deg = jnp.sum(degp_ref[...], axis=0, keepdims=True)
    o_ref[...] = jnp.where(
        deg > 0, 1.0 / jnp.sqrt(jnp.maximum(deg, 1e-12)), 0.0)


def _dis_tc(degp):
    return pl.pallas_call(
        _dis_kern,
        out_shape=jax.ShapeDtypeStruct((1, N_NODES), jnp.float32),
    )(degp)


def _rowsum_kern(a_ref, o_ref):
    o_ref[...] = jnp.sum(a_ref[...], axis=0, keepdims=True)


def _rowsum_tc(a):
    return pl.pallas_call(
        _rowsum_kern,
        out_shape=jax.ShapeDtypeStruct((1, a.shape[1]), jnp.float32),
    )(a)


def _addpair_kern(a_ref, o_ref):
    o_ref[...] = a_ref[0] + a_ref[1]


def _addpair_tc(a):
    return pl.pallas_call(
        _addpair_kern,
        out_shape=jax.ShapeDtypeStruct(a.shape[1:], jnp.float32),
    )(a)


def _gnorm_relu(z, w_ref, b_ref, ms_ref):
    mean = jnp.mean(z, axis=0, keepdims=True)
    out = z - ms_ref[...] * mean
    var = jnp.mean(out * out, axis=0, keepdims=True)
    return jax.nn.relu(w_ref[...] * out / jnp.sqrt(var + 1e-5) + b_ref[...])


def _cheb1_kern(x_ref, t1_ref, t2_ref, w_ref, b_ref, gw_ref, gb_ref, gm_ref,
                o_ref):
    x = x_ref[...]
    t1 = t1_ref[...]
    t2 = t2_ref[...]
    z = (x * w_ref[0] + t1 * w_ref[1] + (2.0 * t2 - x) * w_ref[2] + b_ref[...])
    o_ref[...] = _gnorm_relu(z, gw_ref, gb_ref, gm_ref)


def _cheb1_tc(x, t1c, t2c, w, b, gw, gb, gm):
    return pl.pallas_call(
        _cheb1_kern,
        out_shape=jax.ShapeDtypeStruct((N_NODES, HIDDEN), jnp.float32),
    )(x, t1c, t2c, w, b.reshape(1, HIDDEN), gw.reshape(1, HIDDEN),
      gb.reshape(1, HIDDEN), gm.reshape(1, HIDDEN))


def _cheb23_kern(h_ref, w1_ref, w2p_ref, w_ref, b_ref, gw_ref, gb_ref, gm_ref,
                 o_ref):
    h = h_ref[...]
    w1 = w1_ref[...]
    w2 = w2p_ref[0] + w2p_ref[1]
    tx2 = 2.0 * w2 - h
    dot = functools.partial(
        jax.lax.dot_general,
        dimension_numbers=(((1,), (0,)), ((), ())),
        preferred_element_type=jnp.float32)
    z = dot(h, w_ref[0]) + dot(w1, w_ref[1]) + dot(tx2, w_ref[2]) + b_ref[...]
    o_ref[...] = _gnorm_relu(z, gw_ref, gb_ref, gm_ref)


def _cheb23_tc(h, w1, w2p, w, b, gw, gb, gm):
    return pl.pallas_call(
        _cheb23_kern,
        out_shape=jax.ShapeDtypeStruct((N_NODES, HIDDEN), jnp.float32),
    )(h, w1, w2p, w, b.reshape(1, HIDDEN), gw.reshape(1, HIDDEN),
      gb.reshape(1, HIDDEN), gm.reshape(1, HIDDEN))


def _cheb3_kern(h_ref, w1_ref, w2p_ref, w_ref, b_ref, gw_ref, gb_ref, gm_ref,
                lw_ref, lb_ref, o_ref):
    h = h_ref[...]
    w1 = w1_ref[...]
    w2 = w2p_ref[0] + w2p_ref[1]
    tx2 = 2.0 * w2 - h
    dot = functools.partial(
        jax.lax.dot_general,
        dimension_numbers=(((1,), (0,)), ((), ())),
        preferred_element_type=jnp.float32)
    z = dot(h, w_ref[0]) + dot(w1, w_ref[1]) + dot(tx2, w_ref[2]) + b_ref[...]
    h3 = _gnorm_relu(z, gw_ref, gb_ref, gm_ref)
    gpool = jnp.max(h3, axis=0, keepdims=True)
    o_ref[...] = dot(gpool, lw_ref[...]) + lb_ref[...]


def _cheb3_tc(h, w1, w2p, w, b, gw, gb, gm, lin_w, lin_b):
    return pl.pallas_call(
        _cheb3_kern,
        out_shape=jax.ShapeDtypeStruct((1, NUM_CLASSES), jnp.float32),
    )(h, w1, w2p, w, b.reshape(1, HIDDEN), gw.reshape(1, HIDDEN),
      gb.reshape(1, HIDDEN), gm.reshape(1, HIDDEN), lin_w,
      lin_b.reshape(1, NUM_CLASSES))


def kernel(x, edge_index, order, conv1_w, conv1_b, conv2_w, conv2_b, conv3_w,
           conv3_b, gn1_w, gn1_b, gn1_ms, gn2_w, gn2_b, gn2_ms, gn3_w, gn3_b,
           gn3_ms, lin_w, lin_b):
    row = edge_index[0]
    col = edge_index[1]
    x1d = x.reshape(N_NODES)

    Y = _frft_y(x, order)

    ew, degp = _k1(Y, row, col)
    dis = _dis_tc(degp)
    lapw, t1p = _k2(dis.reshape(N_NODES), x1d, ew, row, col)
    t1 = _rowsum_tc(t1p)
    t2p = _k3(t1.reshape(N_NODES), lapw, row, col)
    t2 = _rowsum_tc(t2p)

    h1 = _cheb1_tc(x, t1.reshape(N_NODES, 1), t2.reshape(N_NODES, 1),
                   conv1_w, conv1_b, gn1_w, gn1_b, gn1_ms)

    w1p = _k4(h1, lapw, row, col)
    w1 = _addpair_tc(w1p)
    w2p = _k4(w1, lapw, row, col)
    h2 = _cheb23_tc(h1, w1, w2p, conv2_w, conv2_b, gn2_w, gn2_b, gn2_ms)

    v1p = _k4(h2, lapw, row, col)
    v1 = _addpair_tc(v1p)
    v2p = _k4(v1, lapw, row, col)
    return _cheb3_tc(h2, v1, v2p, conv3_w, conv3_b, gn3_w, gn3_b, gn3_ms,
                     lin_w, lin_b)


# speedup vs baseline: 21.8224x; 1.4111x over previous
import functools
import math

import jax
import jax.numpy as jnp
import numpy as np
from jax import lax
from jax.experimental import pallas as pl
from jax.experimental.pallas import tpu as pltpu
from jax.experimental.pallas import tpu_sc as plsc

N_NODES = 4096
N_EDGES = 131072
HIDDEN = 128
NUM_CLASSES = 8

NW = 32
EPW = N_EDGES // NW
CHUNK = 256
NPS = N_NODES // 16


def _build_constants():
    N = N_NODES
    n = np.arange(N)
    C = -2.0 * np.eye(N) + np.eye(N, k=1) + np.eye(N, k=-1)
    C[0, N - 1] += 1.0
    C[N - 1, 0] += 1.0
    S = C + np.diag(2.0 * np.cos(2.0 * np.pi * n / N) - 2.0)
    _, evecs = np.linalg.eigh(S)
    E = np.ascontiguousarray(evecs[:, ::-1]).astype(np.float32)
    idxv = np.concatenate([np.arange(N - 1), [N if N % 2 == 0 else N - 1]]).astype(
        np.float32
    )
    cut = max(1, int(round(0.15 * N)))
    sel = np.concatenate([np.arange(cut), np.arange(N - cut, N)])
    R = E[sel, :]
    RPAD = 1280
    Rp = np.zeros((RPAD, N), np.float32)
    Rp[: R.shape[0]] = R
    ET = np.ascontiguousarray(E.T)
    RT = np.ascontiguousarray(Rp.T)
    return E, ET, Rp, RT, idxv, RPAD


_E, _ET, _R, _RT, _IDXV, _RPAD = _build_constants()


_BLK = 512


def _mv_kern(a_ref, b_ref, o_ref):
    o_ref[...] = jax.lax.dot_general(
        a_ref[...], b_ref[...], (((1,), (0,)), ((), ())),
        preferred_element_type=jnp.float32)


def _matmul(a, b, blk):
    m, k = a.shape
    _, nn = b.shape
    return pl.pallas_call(
        _mv_kern,
        grid=(m // blk,),
        in_specs=[
            pl.BlockSpec((blk, k), lambda i: (i, 0)),
            pl.BlockSpec((k, nn), lambda i: (0, 0)),
        ],
        out_specs=pl.BlockSpec((blk, nn), lambda i: (i, 0)),
        out_shape=jax.ShapeDtypeStruct((m, nn), jnp.float32),
    )(a, b)


def _s2_kern(r_ref, t_ref, c_ref, s_ref, o_ref):
    t = t_ref[...]
    s = jnp.concatenate([t * c_ref[...], -t * s_ref[...]], axis=1)
    o_ref[...] = jax.lax.dot_general(
        r_ref[...], s, (((1,), (0,)), ((), ())),
        preferred_element_type=jnp.float32)


def _s4_kern(e_ref, u_ref, c_ref, s_ref, o_ref):
    u = u_ref[...]
    c = c_ref[...]
    s = s_ref[...]
    u0 = u[:, 0:1]
    u1 = u[:, 1:2]
    q = jnp.concatenate([u0 * c - u1 * s, u0 * s + u1 * c], axis=1)
    ya = jax.lax.dot_general(
        e_ref[...], q, (((1,), (0,)), ((), ())),
        preferred_element_type=jnp.float32)
    o_ref[...] = jnp.sqrt(ya[:, 0:1] * ya[:, 0:1] + ya[:, 1:2] * ya[:, 1:2])


def _frft_y(x, order):
    N = N_NODES
    E = jnp.asarray(_E)
    ET = jnp.asarray(_ET)
    R = jnp.asarray(_R)
    RT = jnp.asarray(_RT)
    phi = ((math.pi / 2.0) * order) * jnp.asarray(_IDXV)
    c = jnp.cos(phi).reshape(N, 1)
    s = jnp.sin(phi).reshape(N, 1)

    t = _matmul(ET, x, _BLK)
    w = pl.pallas_call(
        _s2_kern,
        grid=(_RPAD // 320,),
        in_specs=[
            pl.BlockSpec((320, N), lambda i: (i, 0)),
            pl.BlockSpec((N, 1), lambda i: (0, 0)),
            pl.BlockSpec((N, 1), lambda i: (0, 0)),
            pl.BlockSpec((N, 1), lambda i: (0, 0)),
        ],
        out_specs=pl.BlockSpec((320, 2), lambda i: (i, 0)),
        out_shape=jax.ShapeDtypeStruct((_RPAD, 2), jnp.float32),
    )(R, t, c, s)
    u = _matmul(RT, w, _BLK)
    y = pl.pallas_call(
        _s4_kern,
        grid=(N // _BLK,),
        in_specs=[
            pl.BlockSpec((_BLK, N), lambda i: (i, 0)),
            pl.BlockSpec((N, 2), lambda i: (0, 0)),
            pl.BlockSpec((N, 1), lambda i: (0, 0)),
            pl.BlockSpec((N, 1), lambda i: (0, 0)),
        ],
        out_specs=pl.BlockSpec((_BLK, 1), lambda i: (i, 0)),
        out_shape=jax.ShapeDtypeStruct((N, 1), jnp.float32),
    )(E, u, c, s)
    return y.reshape(N)


_MESH = plsc.VectorSubcoreMesh(core_axis_name="c", subcore_axis_name="s")
_SC_PARAMS = pltpu.CompilerParams(needs_layout_passes=False)


def _wid():
    return lax.axis_index("c") * 16 + lax.axis_index("s")


def _lane_iota():
    return lax.iota(jnp.int32, 16)


def _zero_lanes(lanes_v):
    z = jnp.zeros((16,), jnp.float32)

    def body(i, _):
        for l in range(16):
            lanes_v[l, pl.ds(i * 16, 16)] = z
        return 0

    lax.fori_loop(0, N_NODES // 16, body, 0)


def _reduce_lanes(lanes_v, red_v):
    def body(i, _):
        acc = lanes_v[0, pl.ds(i * 16, 16)]
        for l in range(1, 16):
            acc = acc + lanes_v[l, pl.ds(i * 16, 16)]
        red_v[pl.ds(i * 16, 16)] = acc
        return 0

    lax.fori_loop(0, N_NODES // 16, body, 0)


@functools.partial(
    pl.kernel,
    out_type=[
        jax.ShapeDtypeStruct((N_EDGES,), jnp.float32),
        jax.ShapeDtypeStruct((NW, N_NODES), jnp.float32),
    ],
    mesh=_MESH,
    compiler_params=_SC_PARAMS,
    scratch_types=[
        pltpu.VMEM((N_NODES,), jnp.float32),
        pltpu.VMEM((EPW,), jnp.int32),
        pltpu.VMEM((EPW,), jnp.int32),
        pltpu.VMEM((EPW,), jnp.float32),
        pltpu.VMEM((16, N_NODES), jnp.float32),
        pltpu.VMEM((N_NODES,), jnp.float32),
    ],
)
def _k1(y_hbm, row_hbm, col_hbm, ew_hbm, degp_hbm,
        y_v, row_v, col_v, ew_v, lanes_v, red_v):
    w = _wid()
    base = w * EPW
    pltpu.sync_copy(y_hbm, y_v)
    pltpu.sync_copy(row_hbm.at[pl.ds(base, EPW)], row_v)
    pltpu.sync_copy(col_hbm.at[pl.ds(base, EPW)], col_v)
    _zero_lanes(lanes_v)
    li = _lane_iota()

    def body(g, _):
        r16 = row_v[pl.ds(g * 16, 16)]
        c16 = col_v[pl.ds(g * 16, 16)]
        yr = plsc.load_gather(y_v, [r16])
        yc = plsc.load_gather(y_v, [c16])
        d = jnp.abs(yr - yc) / (jnp.abs(yr) + jnp.abs(yc) + 1e-8)
        ew16 = jnp.maximum(1.0 - d, 1e-6)
        ew_v[pl.ds(g * 16, 16)] = ew16
        plsc.addupdate_scatter(lanes_v, [li, r16], ew16)
        return 0

    lax.fori_loop(0, EPW // 16, body, 0)
    _reduce_lanes(lanes_v, red_v)
    pltpu.sync_copy(ew_v, ew_hbm.at[pl.ds(base, EPW)])
    pltpu.sync_copy(red_v, degp_hbm.at[w])


@functools.partial(
    pl.kernel,
    out_type=[
        jax.ShapeDtypeStruct((N_EDGES,), jnp.float32),
        jax.ShapeDtypeStruct((NW, N_NODES), jnp.float32),
    ],
    mesh=_MESH,
    compiler_params=_SC_PARAMS,
    scratch_types=[
        pltpu.VMEM((N_NODES,), jnp.float32),
        pltpu.VMEM((N_NODES,), jnp.float32),
        pltpu.VMEM((EPW,), jnp.int32),
        pltpu.VMEM((EPW,), jnp.int32),
        pltpu.VMEM((EPW,), jnp.float32),
        pltpu.VMEM((EPW,), jnp.float32),
        pltpu.VMEM((16, N_NODES), jnp.float32),
        pltpu.VMEM((N_NODES,), jnp.float32),
    ],
)
def _k2(dis_hbm, x_hbm, ew_hbm, row_hbm, col_hbm, lapw_hbm, t1p_hbm,
        dis_v, x_v, row_v, col_v, ew_v, lapw_v, lanes_v, red_v):
    w = _wid()
    base = w * EPW
    pltpu.sync_copy(dis_hbm, dis_v)
    pltpu.sync_copy(x_hbm, x_v)
    pltpu.sync_copy(row_hbm.at[pl.ds(base, EPW)], row_v)
    pltpu.sync_copy(col_hbm.at[pl.ds(base, EPW)], col_v)
    pltpu.sync_copy(ew_hbm.at[pl.ds(base, EPW)], ew_v)
    _zero_lanes(lanes_v)
    li = _lane_iota()

    def body(g, _):
        r16 = row_v[pl.ds(g * 16, 16)]
        c16 = col_v[pl.ds(g * 16, 16)]
        ew16 = ew_v[pl.ds(g * 16, 16)]
        dr = plsc.load_gather(dis_v, [r16])
        dc = plsc.load_gather(dis_v, [c16])
        lw16 = -dr * ew16 * dc
        lapw_v[pl.ds(g * 16, 16)] = lw16
        xr = plsc.load_gather(x_v, [r16])
        plsc.addupdate_scatter(lanes_v, [li, c16], lw16 * xr)
        return 0

    lax.fori_loop(0, EPW // 16, body, 0)
    _reduce_lanes(lanes_v, red_v)
    pltpu.sync_copy(lapw_v, lapw_hbm.at[pl.ds(base, EPW)])
    pltpu.sync_copy(red_v, t1p_hbm.at[w])


@functools.partial(
    pl.kernel,
    out_type=jax.ShapeDtypeStruct((NW, N_NODES), jnp.float32),
    mesh=_MESH,
    compiler_params=_SC_PARAMS,
    scratch_types=[
        pltpu.VMEM((N_NODES,), jnp.float32),
        pltpu.VMEM((EPW,), jnp.int32),
        pltpu.VMEM((EPW,), jnp.int32),
        pltpu.VMEM((EPW,), jnp.float32),
        pltpu.VMEM((16, N_NODES), jnp.float32),
        pltpu.VMEM((N_NODES,), jnp.float32),
    ],
)
def _k3(t1_hbm, lapw_hbm, row_hbm, col_hbm, t2p_hbm,
        v_v, row_v, col_v, lapw_v, lanes_v, red_v):
    w = _wid()
    base = w * EPW
    pltpu.sync_copy(t1_hbm, v_v)
    pltpu.sync_copy(row_hbm.at[pl.ds(base, EPW)], row_v)
    pltpu.sync_copy(col_hbm.at[pl.ds(base, EPW)], col_v)
    pltpu.sync_copy(lapw_hbm.at[pl.ds(base, EPW)], lapw_v)
    _zero_lanes(lanes_v)
    li = _lane_iota()

    def body(g, _):
        r16 = row_v[pl.ds(g * 16, 16)]
        c16 = col_v[pl.ds(g * 16, 16)]
        lw16 = lapw_v[pl.ds(g * 16, 16)]
        vr = plsc.load_gather(v_v, [r16])
        plsc.addupdate_scatter(lanes_v, [li, c16], lw16 * vr)
        return 0

    lax.fori_loop(0, EPW // 16, body, 0)
    _reduce_lanes(lanes_v, red_v)
    pltpu.sync_copy(red_v, t2p_hbm.at[w])


_NCH = EPW // CHUNK


@functools.partial(
    pl.kernel,
    out_type=jax.ShapeDtypeStruct((2, N_NODES, HIDDEN), jnp.float32),
    mesh=_MESH,
    compiler_params=_SC_PARAMS,
    scratch_types=[
        pltpu.VMEM((EPW,), jnp.int32),
        pltpu.VMEM((EPW,), jnp.int32),
        pltpu.VMEM((EPW,), jnp.float32),
        pltpu.VMEM((CHUNK, HIDDEN), jnp.float32),
        pltpu.VMEM((CHUNK, HIDDEN), jnp.float32),
        pltpu.VMEM_SHARED((N_NODES, HIDDEN), jnp.float32),
        pltpu.SemaphoreType.DMA,
        pltpu.SemaphoreType.DMA,
        pltpu.SemaphoreType.DMA,
        pltpu.SemaphoreType.DMA,
    ],
)
def _k4(h_hbm, lapw_hbm, row_hbm, col_hbm, out_hbm,
        row_v, col_v, lw_v, rows0_v, rows1_v, acc_sh,
        gsem0, gsem1, ssem0, ssem1):
    cid = lax.axis_index("c")
    sid = lax.axis_index("s")
    w = cid * 16 + sid
    base = w * EPW
    rows_b = (rows0_v, rows1_v)
    gsem = (gsem0, gsem1)
    ssem = (ssem0, ssem1)
    zrow = jnp.zeros((16,), jnp.float32)

    pltpu.sync_copy(row_hbm.at[pl.ds(base, EPW)], row_v)
    pltpu.sync_copy(col_hbm.at[pl.ds(base, EPW)], col_v)
    pltpu.sync_copy(lapw_hbm.at[pl.ds(base, EPW)], lw_v)

    def zbody(i, _):
        for j in range(8):
            rows0_v[i, pl.ds(j * 16, 16)] = zrow
        return 0

    lax.fori_loop(0, CHUNK, zbody, 0)
    pltpu.sync_copy(rows0_v, acc_sh.at[pl.ds(sid * NPS, NPS)])
    plsc.subcore_barrier()

    def scale(b, g):
        def scale_body(e, _):
            ei = jnp.zeros((16,), jnp.int32) + (g * CHUNK + e)
            lw = plsc.load_gather(lw_v, [ei])
            for j in range(8):
                b[e, pl.ds(j * 16, 16)] = b[e, pl.ds(j * 16, 16)] * lw
            return 0

        lax.fori_loop(0, CHUNK, scale_body, 0)

    gd = [None, None]
    sd = [None, None]
    gd[0] = pltpu.async_copy(
        h_hbm.at[row_v.at[pl.ds(0, CHUNK)]], rows0_v, gsem[0])
    for g in range(_NCH):
        b = g & 1
        gd[b].wait()
        if g + 1 < _NCH:
            nb = (g + 1) & 1
            if sd[nb] is not None:
                sd[nb].wait()
            gd[nb] = pltpu.async_copy(
                h_hbm.at[row_v.at[pl.ds((g + 1) * CHUNK, CHUNK)]],
                rows_b[nb], gsem[nb])
        scale(rows_b[b], g)
        sd[b] = pltpu.async_copy(
            rows_b[b], acc_sh.at[col_v.at[pl.ds(g * CHUNK, CHUNK)]],
            ssem[b], add=True)
    sd[0].wait()
    sd[1].wait()
    plsc.subcore_barrier()
    pltpu.sync_copy(acc_sh.at[pl.ds(sid * NPS, NPS)],
                    out_hbm.at[cid, pl.ds(sid * NPS, NPS)])


def _dis_kern(degp_ref, o_ref):
    deg = jnp.sum(degp_ref[...], axis=0, keepdims=True)
    o_ref[...] = jnp.where(
        deg > 0, 1.0 / jnp.sqrt(jnp.maximum(deg, 1e-12)), 0.0)


def _dis_tc(degp):
    return pl.pallas_call(
        _dis_kern,
        out_shape=jax.ShapeDtypeStruct((1, N_NODES), jnp.float32),
    )(degp)


def _rowsum_kern(a_ref, o_ref):
    o_ref[...] = jnp.sum(a_ref[...], axis=0, keepdims=True)


def _rowsum_tc(a):
    return pl.pallas_call(
        _rowsum_kern,
        out_shape=jax.ShapeDtypeStruct((1, a.shape[1]), jnp.float32),
    )(a)


def _addpair_kern(a_ref, o_ref):
    o_ref[...] = a_ref[0] + a_ref[1]


def _addpair_tc(a):
    return pl.pallas_call(
        _addpair_kern,
        out_shape=jax.ShapeDtypeStruct(a.shape[1:], jnp.float32),
    )(a)


def _gnorm_relu(z, w_ref, b_ref, ms_ref):
    mean = jnp.mean(z, axis=0, keepdims=True)
    out = z - ms_ref[...] * mean
    var = jnp.mean(out * out, axis=0, keepdims=True)
    return jax.nn.relu(w_ref[...] * out / jnp.sqrt(var + 1e-5) + b_ref[...])


def _cheb1_kern(x_ref, t1_ref, t2_ref, w_ref, b_ref, gw_ref, gb_ref, gm_ref,
                o_ref):
    x = x_ref[...]
    t1 = t1_ref[...]
    t2 = t2_ref[...]
    z = (x * w_ref[0] + t1 * w_ref[1] + (2.0 * t2 - x) * w_ref[2] + b_ref[...])
    o_ref[...] = _gnorm_relu(z, gw_ref, gb_ref, gm_ref)


def _cheb1_tc(x, t1c, t2c, w, b, gw, gb, gm):
    return pl.pallas_call(
        _cheb1_kern,
        out_shape=jax.ShapeDtypeStruct((N_NODES, HIDDEN), jnp.float32),
    )(x, t1c, t2c, w, b.reshape(1, HIDDEN), gw.reshape(1, HIDDEN),
      gb.reshape(1, HIDDEN), gm.reshape(1, HIDDEN))


def _cheb23_kern(h_ref, w1_ref, w2p_ref, w_ref, b_ref, gw_ref, gb_ref, gm_ref,
                 o_ref):
    h = h_ref[...]
    w1 = w1_ref[...]
    w2 = w2p_ref[0] + w2p_ref[1]
    tx2 = 2.0 * w2 - h
    dot = functools.partial(
        jax.lax.dot_general,
        dimension_numbers=(((1,), (0,)), ((), ())),
        preferred_element_type=jnp.float32)
    z = dot(h, w_ref[0]) + dot(w1, w_ref[1]) + dot(tx2, w_ref[2]) + b_ref[...]
    o_ref[...] = _gnorm_relu(z, gw_ref, gb_ref, gm_ref)


def _cheb23_tc(h, w1, w2p, w, b, gw, gb, gm):
    return pl.pallas_call(
        _cheb23_kern,
        out_shape=jax.ShapeDtypeStruct((N_NODES, HIDDEN), jnp.float32),
    )(h, w1, w2p, w, b.reshape(1, HIDDEN), gw.reshape(1, HIDDEN),
      gb.reshape(1, HIDDEN), gm.reshape(1, HIDDEN))


def _cheb3_kern(h_ref, w1_ref, w2p_ref, w_ref, b_ref, gw_ref, gb_ref, gm_ref,
                lw_ref, lb_ref, o_ref):
    h = h_ref[...]
    w1 = w1_ref[...]
    w2 = w2p_ref[0] + w2p_ref[1]
    tx2 = 2.0 * w2 - h
    dot = functools.partial(
        jax.lax.dot_general,
        dimension_numbers=(((1,), (0,)), ((), ())),
        preferred_element_type=jnp.float32)
    z = dot(h, w_ref[0]) + dot(w1, w_ref[1]) + dot(tx2, w_ref[2]) + b_ref[...]
    h3 = _gnorm_relu(z, gw_ref, gb_ref, gm_ref)
    gpool = jnp.max(h3, axis=0, keepdims=True)
    o_ref[...] = dot(gpool, lw_ref[...]) + lb_ref[...]


def _cheb3_tc(h, w1, w2p, w, b, gw, gb, gm, lin_w, lin_b):
    return pl.pallas_call(
        _cheb3_kern,
        out_shape=jax.ShapeDtypeStruct((1, NUM_CLASSES), jnp.float32),
    )(h, w1, w2p, w, b.reshape(1, HIDDEN), gw.reshape(1, HIDDEN),
      gb.reshape(1, HIDDEN), gm.reshape(1, HIDDEN), lin_w,
      lin_b.reshape(1, NUM_CLASSES))


def kernel(x, edge_index, order, conv1_w, conv1_b, conv2_w, conv2_b, conv3_w,
           conv3_b, gn1_w, gn1_b, gn1_ms, gn2_w, gn2_b, gn2_ms, gn3_w, gn3_b,
           gn3_ms, lin_w, lin_b):
    row = edge_index[0]
    col = edge_index[1]
    x1d = x.reshape(N_NODES)

    Y = _frft_y(x, order)

    ew, degp = _k1(Y, row, col)
    dis = _dis_tc(degp)
    lapw, t1p = _k2(dis.reshape(N_NODES), x1d, ew, row, col)
    t1 = _rowsum_tc(t1p)
    t2p = _k3(t1.reshape(N_NODES), lapw, row, col)
    t2 = _rowsum_tc(t2p)

    h1 = _cheb1_tc(x, t1.reshape(N_NODES, 1), t2.reshape(N_NODES, 1),
                   conv1_w, conv1_b, gn1_w, gn1_b, gn1_ms)

    w1p = _k4(h1, lapw, row, col)
    w1 = _addpair_tc(w1p)
    w2p = _k4(w1, lapw, row, col)
    h2 = _cheb23_tc(h1, w1, w2p, conv2_w, conv2_b, gn2_w, gn2_b, gn2_ms)

    v1p = _k4(h2, lapw, row, col)
    v1 = _addpair_tc(v1p)
    v2p = _k4(v1, lapw, row, col)
    return _cheb3_tc(h2, v1, v2p, conv3_w, conv3_b, gn3_w, gn3_b, gn3_ms,
                     lin_w, lin_b)
